# baseline (speedup 1.0000x reference)
import dataclasses
import functools

import jax
import jax.numpy as jnp
from jax import lax
from jax.experimental import pallas as pl
from jax.experimental.pallas import tpu as pltpu
from jax.experimental.pallas import tpu_sc as plsc

NUM_CORES = 2
NUM_SUBCORES = 16
NUM_WORKERS = NUM_CORES * NUM_SUBCORES
LANES = 16


def _score_sc(x, src, dst, n_edges, d_feat):
    e_per_w = n_edges // NUM_WORKERS
    chunk = 80
    n_chunks = e_per_w // chunk
    n_fvec = d_feat // LANES

    mesh = plsc.VectorSubcoreMesh(core_axis_name="c", subcore_axis_name="s")

    cp = pltpu.CompilerParams()
    if "needs_layout_passes" in pltpu.CompilerParams.__dataclass_fields__:
        cp = dataclasses.replace(cp, needs_layout_passes=False)

    @functools.partial(
        pl.kernel,
        compiler_params=cp,
        out_type=jax.ShapeDtypeStruct((n_edges,), jnp.float32),
        mesh=mesh,
        scratch_types=[
            pltpu.VMEM((e_per_w,), jnp.int32),
            pltpu.VMEM((e_per_w,), jnp.int32),
            pltpu.VMEM((chunk, d_feat), jnp.float32),
            pltpu.VMEM((chunk, d_feat), jnp.float32),
            pltpu.VMEM((chunk, d_feat), jnp.float32),
            pltpu.VMEM((chunk, d_feat), jnp.float32),
            pltpu.VMEM((chunk,), jnp.float32),
            pltpu.VMEM((chunk,), jnp.float32),
            pltpu.SemaphoreType.DMA,
            pltpu.SemaphoreType.DMA,
        ],
    )
    def sc_kernel(x_hbm, src_hbm, dst_hbm, out_hbm, sidx, didx,
                  u0, v0, u1, v1, s0, s1, sem0, sem1):
        wid = lax.axis_index("s") * NUM_CORES + lax.axis_index("c")
        base = wid * e_per_w
        pltpu.sync_copy(src_hbm.at[pl.ds(base, e_per_w)], sidx)
        pltpu.sync_copy(dst_hbm.at[pl.ds(base, e_per_w)], didx)

        lane_iota = lax.broadcasted_iota(jnp.int32, (LANES,), 0)

        def fire(ci, u, v, sem):
            off = ci * chunk
            pltpu.async_copy(x_hbm.at[sidx.at[pl.ds(off, chunk)]], u, sem)
            pltpu.async_copy(x_hbm.at[didx.at[pl.ds(off, chunk)]], v, sem)

        def drain(u, v, sem):
            pltpu.make_async_copy(x_hbm.at[sidx.at[pl.ds(0, chunk)]], u,
                                  sem).wait()
            pltpu.make_async_copy(x_hbm.at[didx.at[pl.ds(0, chunk)]], v,
                                  sem).wait()

        def row_dot(u, v, r):
            acc = u[r, pl.ds(0, LANES)] * v[r, pl.ds(0, LANES)]
            for c in range(1, n_fvec):
                acc = acc + (u[r, pl.ds(c * LANES, LANES)] *
                             v[r, pl.ds(c * LANES, LANES)])
            return jnp.sum(acc)

        def compute(ci, u, v, s):
            for g in range(chunk // LANES):
                vec = jnp.zeros((LANES,), jnp.float32)
                for j in range(LANES):
                    d = row_dot(u, v, g * LANES + j)
                    vec = jnp.where(lane_iota == j, d, vec)
                s[pl.ds(g * LANES, LANES)] = vec

            pltpu.sync_copy(s, out_hbm.at[pl.ds(base + ci * chunk, chunk)])

        fire(0, u0, v0, sem0)

        @pl.loop(0, n_chunks - 1, step=2)
        def _chunk_body(i):
            fire(i + 1, u1, v1, sem1)
            drain(u0, v0, sem0)
            compute(i, u0, v0, s0)
            fire(i + 2, u0, v0, sem0)
            drain(u1, v1, sem1)
            compute(i + 1, u1, v1, s1)

        drain(u0, v0, sem0)
        compute(n_chunks - 1, u0, v0, s0)

    return sc_kernel(x, src, dst)


def kernel(x, edge_index):
    n_edges = edge_index.shape[1]
    d_feat = x.shape[1]
    src = edge_index[0]
    dst = edge_index[1]
    score = _score_sc(x, src, dst, n_edges, d_feat)
    return score.reshape(n_edges, 1)


# speedup vs baseline: 2.8564x; 2.8564x over previous
import dataclasses
import functools

import jax
import jax.numpy as jnp
from jax import lax
from jax.experimental import pallas as pl
from jax.experimental.pallas import tpu as pltpu
from jax.experimental.pallas import tpu_sc as plsc

NUM_CORES = 2
NUM_SUBCORES = 16
NUM_WORKERS = NUM_CORES * NUM_SUBCORES
LANES = 16


def _score_sc(x, src, dst, n_edges, d_feat):
    e_per_w = n_edges // NUM_WORKERS
    chunk = 80
    n_chunks = e_per_w // chunk
    n_fvec = d_feat // LANES

    mesh = plsc.VectorSubcoreMesh(core_axis_name="c", subcore_axis_name="s")

    cp = pltpu.CompilerParams()
    if "needs_layout_passes" in pltpu.CompilerParams.__dataclass_fields__:
        cp = dataclasses.replace(cp, needs_layout_passes=False)

    @functools.partial(
        pl.kernel,
        compiler_params=cp,
        out_type=jax.ShapeDtypeStruct((n_edges,), jnp.float32),
        mesh=mesh,
        scratch_types=[
            pltpu.VMEM((e_per_w,), jnp.int32),
            pltpu.VMEM((e_per_w,), jnp.int32),
            pltpu.VMEM((chunk, d_feat), jnp.float32),
            pltpu.VMEM((chunk, d_feat), jnp.float32),
            pltpu.VMEM((chunk, d_feat), jnp.float32),
            pltpu.VMEM((chunk, d_feat), jnp.float32),
            pltpu.VMEM((chunk,), jnp.float32),
            pltpu.VMEM((chunk,), jnp.float32),
            pltpu.SemaphoreType.DMA,
            pltpu.SemaphoreType.DMA,
        ],
    )
    def sc_kernel(x_hbm, src_hbm, dst_hbm, out_hbm, sidx, didx,
                  u0, v0, u1, v1, s0, s1, sem0, sem1):
        wid = lax.axis_index("s") * NUM_CORES + lax.axis_index("c")
        base = wid * e_per_w
        pltpu.sync_copy(src_hbm.at[pl.ds(base, e_per_w)], sidx)
        pltpu.sync_copy(dst_hbm.at[pl.ds(base, e_per_w)], didx)

        lane_iota = lax.broadcasted_iota(jnp.int32, (LANES,), 0)

        def fire(ci, u, v, sem):
            off = ci * chunk
            pltpu.async_copy(x_hbm.at[sidx.at[pl.ds(off, chunk)]], u, sem)
            pltpu.async_copy(x_hbm.at[didx.at[pl.ds(off, chunk)]], v, sem)

        def drain(u, v, sem):
            pltpu.make_async_copy(x_hbm.at[sidx.at[pl.ds(0, chunk)]], u,
                                  sem).wait()
            pltpu.make_async_copy(x_hbm.at[didx.at[pl.ds(0, chunk)]], v,
                                  sem).wait()

        def row_dot(u, v, r):
            acc = u[r, pl.ds(0, LANES)] * v[r, pl.ds(0, LANES)]
            for c in range(1, n_fvec):
                acc = acc + (u[r, pl.ds(c * LANES, LANES)] *
                             v[r, pl.ds(c * LANES, LANES)])
            return jnp.sum(acc)

        def compute(ci, u, v, s):
            @pl.loop(0, chunk // LANES)
            def _group_body(g):
                def row_body(j, vec):
                    r = g * LANES + 2 * j
                    d0 = row_dot(u, v, r)
                    d1 = row_dot(u, v, r + 1)
                    vec = jnp.where(lane_iota == 2 * j, d0, vec)
                    return jnp.where(lane_iota == 2 * j + 1, d1, vec)

                s[pl.ds(g * LANES, LANES)] = lax.fori_loop(
                    0, LANES // 2, row_body, jnp.zeros((LANES,), jnp.float32))

            pltpu.sync_copy(s, out_hbm.at[pl.ds(base + ci * chunk, chunk)])

        fire(0, u0, v0, sem0)

        @pl.loop(0, n_chunks - 1, step=2)
        def _chunk_body(i):
            fire(i + 1, u1, v1, sem1)
            drain(u0, v0, sem0)
            compute(i, u0, v0, s0)
            fire(i + 2, u0, v0, sem0)
            drain(u1, v1, sem1)
            compute(i + 1, u1, v1, s1)

        drain(u0, v0, sem0)
        compute(n_chunks - 1, u0, v0, s0)

    return sc_kernel(x, src, dst)


def kernel(x, edge_index):
    n_edges = edge_index.shape[1]
    d_feat = x.shape[1]
    src = edge_index[0]
    dst = edge_index[1]
    score = _score_sc(x, src, dst, n_edges, d_feat)
    return score.reshape(n_edges, 1)
